# baseline (speedup 1.0000x reference)
import functools

import jax
import jax.numpy as jnp
from jax import lax
from jax.experimental import pallas as pl
from jax.experimental.pallas import tpu as pltpu
from jax.experimental.pallas import tpu_sc as plsc


def _sc_fetch(stab_id, cycle_id, stab_emb_t, cycle_emb, val_emb):
    (L,) = stab_id.shape
    D = stab_emb_t.shape[0]
    info = plsc.get_sparse_core_info()
    NC, NS = info.num_cores, info.num_subcores
    NW = NC * NS
    RPW = 8
    n_active = L // RPW
    assert n_active * RPW == L and n_active <= NW

    mesh = plsc.VectorSubcoreMesh(core_axis_name="c", subcore_axis_name="s")

    @functools.partial(
        pl.kernel,
        out_type=(
            jax.ShapeDtypeStruct((L, D, 128), jnp.float32),
            jax.ShapeDtypeStruct((L, D), jnp.float32),
        ),
        mesh=mesh,
        scratch_types=[
            pltpu.VMEM((16,), jnp.int32),
            pltpu.VMEM((16,), jnp.int32),
            pltpu.VMEM((RPW * D, 128), jnp.float32),
            pltpu.VMEM((RPW, D), jnp.float32),
            pltpu.VMEM((2, D), jnp.float32),
            pltpu.VMEM((RPW, D), jnp.float32),
            pltpu.SemaphoreType.DMA,
            pltpu.SemaphoreType.DMA,
        ],
    )
    def sc_kernel(stab_id_hbm, cycle_id_hbm, stab_t_hbm, cycle_emb_hbm,
                  val_emb_hbm, tiles_hbm, c0_hbm,
                  sidx_v, cidx_v, stile_v, crows_v, val_v, o0_v,
                  sem0, sem1):
        wid = lax.axis_index("s") * NC + lax.axis_index("c")

        @pl.when(wid < n_active)
        def _():
            base = wid * RPW
            pltpu.sync_copy(stab_id_hbm.at[pl.ds(base, RPW)],
                            sidx_v.at[pl.ds(0, RPW)])
            pltpu.sync_copy(cycle_id_hbm.at[pl.ds(base, RPW)],
                            cidx_v.at[pl.ds(0, RPW)])
            pltpu.sync_copy(val_emb_hbm, val_v)
            siv = sidx_v[...]
            civ = cidx_v[...]
            for r in range(RPW):
                off = pl.multiple_of((siv[r] // 128) * 128, 128)
                pltpu.async_copy(stab_t_hbm.at[:, pl.ds(off, 128)],
                                 stile_v.at[pl.ds(r * D, D)], sem0)
                pltpu.async_copy(cycle_emb_hbm.at[pl.ds(civ[r], 1)],
                                 crows_v.at[pl.ds(r, 1)], sem1)
            for r in range(RPW):
                pltpu.make_async_copy(stab_t_hbm.at[:, pl.ds(0, 128)],
                                      stile_v.at[pl.ds(r * D, D)], sem0).wait()
                pltpu.make_async_copy(cycle_emb_hbm.at[pl.ds(0, 1)],
                                      crows_v.at[pl.ds(r, 1)], sem1).wait()
            for r in range(RPW):
                for c in range(D // 16):
                    sl = pl.ds(c * 16, 16)
                    o0_v[r, sl] = crows_v[r, sl] + val_v[0, sl]
            for r in range(RPW):
                pltpu.async_copy(stile_v.at[pl.ds(r * D, D)],
                                 tiles_hbm.at[base + r], sem0)
            pltpu.sync_copy(o0_v, c0_hbm.at[pl.ds(base, RPW)])
            for r in range(RPW):
                pltpu.make_async_copy(stile_v.at[pl.ds(r * D, D)],
                                      tiles_hbm.at[base + r], sem0).wait()

    return sc_kernel(stab_id, cycle_id, stab_emb_t, cycle_emb, val_emb)


def _tc_extract(tiles, colmod3):
    L, D, W = tiles.shape
    TL = 8
    n_steps = L // TL
    assert colmod3.shape == (n_steps, 1, TL)

    def body(tiles_ref, cm_ref, out_ref):
        cm = cm_ref[0]
        iota = lax.broadcasted_iota(jnp.int32, (1, W), 1)
        for j in range(TL):
            cmb = jnp.broadcast_to(jax.lax.slice(cm, (0, j), (1, j + 1)),
                                   (1, W))
            mask = jnp.broadcast_to(iota == cmb, (D, W))
            sel = jnp.where(mask, tiles_ref[j], 0.0)
            col = jnp.sum(sel, axis=1, keepdims=True)
            out_ref[pl.ds(j, 1)] = jnp.swapaxes(col, 0, 1)

    return pl.pallas_call(
        body,
        grid=(n_steps,),
        in_specs=[
            pl.BlockSpec((TL, D, W), lambda i: (i, 0, 0)),
            pl.BlockSpec((1, 1, TL), lambda i: (i, 0, 0)),
        ],
        out_specs=pl.BlockSpec((TL, D), lambda i: (i, 0)),
        out_shape=jax.ShapeDtypeStruct((L, D), jnp.float32),
    )(tiles, colmod3)


def _tc_expand(syndrome_t, tiles, colmod, c0, val_emb):
    L, B = syndrome_t.shape
    D = c0.shape[1]
    W = tiles.shape[2]
    TB = 256
    assert B % TB == 0

    def body(cm_ref, syn_ref, tiles_ref, c0_ref, val_ref, out_ref,
             rep0, drep):
        @pl.when(pl.program_id(0) == 0)
        def _build():
            v = val_ref[...]
            dv = jnp.swapaxes(jax.lax.slice(v, (1, 0), (2, D))
                              - jax.lax.slice(v, (0, 0), (1, D)), 0, 1)
            drep[...] = jnp.broadcast_to(dv, (D, TB))
            c0t = jnp.swapaxes(c0_ref[...], 0, 1)
            iota = lax.broadcasted_iota(jnp.int32, (W, 1), 0)
            for l in range(L):
                onehot = (iota == cm_ref[l]).astype(jnp.float32)
                scol = lax.dot_general(
                    tiles_ref[l], onehot, (((1,), (0,)), ((), ())),
                    precision=lax.Precision.HIGHEST,
                    preferred_element_type=jnp.float32)
                b0c = scol + jax.lax.slice(c0t, (0, l), (D, l + 1))
                rep0[pl.ds(l, 1)] = jnp.broadcast_to(b0c, (D, TB))[None]

        dv = drep[...][None]
        for l in range(L):
            synf = syn_ref[pl.ds(l, 1), :].astype(jnp.float32)
            sb = jnp.broadcast_to(synf, (D, TB))[None]
            out_ref[pl.ds(l, 1)] = rep0[pl.ds(l, 1)] + sb * dv

    return pl.pallas_call(
        body,
        grid=(B // TB,),
        in_specs=[
            pl.BlockSpec(memory_space=pltpu.SMEM),
            pl.BlockSpec((L, TB), lambda i: (0, i)),
            pl.BlockSpec((L, D, W), lambda i: (0, 0, 0)),
            pl.BlockSpec((L, D), lambda i: (0, 0)),
            pl.BlockSpec((2, D), lambda i: (0, 0)),
        ],
        out_specs=pl.BlockSpec((L, D, TB), lambda i: (0, 0, i)),
        out_shape=jax.ShapeDtypeStruct((L, D, B), jnp.float32),
        scratch_shapes=[
            pltpu.VMEM((L, D, TB), jnp.float32),
            pltpu.VMEM((D, TB), jnp.float32),
        ],
    )(colmod, syndrome_t, tiles, c0, val_emb)


def kernel(syndrome, stab_id, cycle_id, stab_emb, cycle_emb, val_emb):
    syndrome = syndrome.astype(jnp.int32)
    stab_id = stab_id.astype(jnp.int32)
    cycle_id = cycle_id.astype(jnp.int32)
    tiles, c0 = _sc_fetch(stab_id, cycle_id, stab_emb.T, cycle_emb, val_emb)
    colmod = stab_id % 128
    out_t = _tc_expand(syndrome.T, tiles, colmod, c0, val_emb)
    return jnp.transpose(out_t, (2, 0, 1))


# speedup vs baseline: 1.0080x; 1.0080x over previous
import functools

import jax
import jax.numpy as jnp
from jax import lax
from jax.experimental import pallas as pl
from jax.experimental.pallas import tpu as pltpu
from jax.experimental.pallas import tpu_sc as plsc


def _sc_fetch(stab_id, cycle_id, stab_emb_t, cycle_emb, val_emb):
    (L,) = stab_id.shape
    D = stab_emb_t.shape[0]
    info = plsc.get_sparse_core_info()
    NC, NS = info.num_cores, info.num_subcores
    NW = NC * NS
    RPW = 8
    n_active = L // RPW
    assert n_active * RPW == L and n_active <= NW

    mesh = plsc.VectorSubcoreMesh(core_axis_name="c", subcore_axis_name="s")

    @functools.partial(
        pl.kernel,
        out_type=(
            jax.ShapeDtypeStruct((L, D, 128), jnp.float32),
            jax.ShapeDtypeStruct((L, D), jnp.float32),
        ),
        mesh=mesh,
        scratch_types=[
            pltpu.VMEM((16,), jnp.int32),
            pltpu.VMEM((16,), jnp.int32),
            pltpu.VMEM((RPW * D, 128), jnp.float32),
            pltpu.VMEM((RPW, D), jnp.float32),
            pltpu.VMEM((2, D), jnp.float32),
            pltpu.VMEM((RPW, D), jnp.float32),
            pltpu.SemaphoreType.DMA,
            pltpu.SemaphoreType.DMA,
        ],
    )
    def sc_kernel(stab_id_hbm, cycle_id_hbm, stab_t_hbm, cycle_emb_hbm,
                  val_emb_hbm, tiles_hbm, c0_hbm,
                  sidx_v, cidx_v, stile_v, crows_v, val_v, o0_v,
                  sem0, sem1):
        wid = lax.axis_index("s") * NC + lax.axis_index("c")

        @pl.when(wid < n_active)
        def _():
            base = wid * RPW
            pltpu.sync_copy(stab_id_hbm.at[pl.ds(base, RPW)],
                            sidx_v.at[pl.ds(0, RPW)])
            pltpu.sync_copy(cycle_id_hbm.at[pl.ds(base, RPW)],
                            cidx_v.at[pl.ds(0, RPW)])
            pltpu.sync_copy(val_emb_hbm, val_v)
            siv = sidx_v[...]
            civ = cidx_v[...]
            for r in range(RPW):
                off = pl.multiple_of((siv[r] // 128) * 128, 128)
                pltpu.async_copy(stab_t_hbm.at[:, pl.ds(off, 128)],
                                 stile_v.at[pl.ds(r * D, D)], sem0)
                pltpu.async_copy(cycle_emb_hbm.at[pl.ds(civ[r], 1)],
                                 crows_v.at[pl.ds(r, 1)], sem1)
            for r in range(RPW):
                pltpu.make_async_copy(stab_t_hbm.at[:, pl.ds(0, 128)],
                                      stile_v.at[pl.ds(r * D, D)], sem0).wait()
                pltpu.make_async_copy(cycle_emb_hbm.at[pl.ds(0, 1)],
                                      crows_v.at[pl.ds(r, 1)], sem1).wait()
            for r in range(RPW):
                for c in range(D // 16):
                    sl = pl.ds(c * 16, 16)
                    o0_v[r, sl] = crows_v[r, sl] + val_v[0, sl]
            for r in range(RPW):
                pltpu.async_copy(stile_v.at[pl.ds(r * D, D)],
                                 tiles_hbm.at[base + r], sem0)
            pltpu.sync_copy(o0_v, c0_hbm.at[pl.ds(base, RPW)])
            for r in range(RPW):
                pltpu.make_async_copy(stile_v.at[pl.ds(r * D, D)],
                                      tiles_hbm.at[base + r], sem0).wait()

    return sc_kernel(stab_id, cycle_id, stab_emb_t, cycle_emb, val_emb)


def _tc_expand(syndrome_t, tiles, colmod, c0, val_emb):
    L, B = syndrome_t.shape
    D = c0.shape[1]
    W = tiles.shape[2]
    TB = 128
    assert B % TB == 0

    def body(cm_ref, syn_ref, tiles_ref, c0_ref, val_ref, out_ref,
             rep0, drep):
        @pl.when(pl.program_id(0) == 0)
        def _build():
            v = val_ref[...]
            dv = jnp.swapaxes(jax.lax.slice(v, (1, 0), (2, D))
                              - jax.lax.slice(v, (0, 0), (1, D)), 0, 1)
            drep[...] = jnp.broadcast_to(dv, (D, TB))
            c0t = jnp.swapaxes(c0_ref[...], 0, 1)
            iota = lax.broadcasted_iota(jnp.int32, (W, 1), 0)
            for l in range(L):
                onehot = (iota == cm_ref[l]).astype(jnp.float32)
                scol = lax.dot_general(
                    tiles_ref[l], onehot, (((1,), (0,)), ((), ())),
                    precision=lax.Precision.HIGHEST,
                    preferred_element_type=jnp.float32)
                b0c = scol + jax.lax.slice(c0t, (0, l), (D, l + 1))
                rep0[pl.ds(l, 1)] = jnp.broadcast_to(b0c, (D, TB))[None]

        dv = drep[...][None]
        for l in range(L):
            synf = syn_ref[pl.ds(l, 1), :].astype(jnp.float32)
            sb = jnp.broadcast_to(synf, (D, TB))[None]
            out_ref[pl.ds(l, 1)] = rep0[pl.ds(l, 1)] + sb * dv

    return pl.pallas_call(
        body,
        grid=(B // TB,),
        in_specs=[
            pl.BlockSpec(memory_space=pltpu.SMEM),
            pl.BlockSpec((L, TB), lambda i: (0, i)),
            pl.BlockSpec((L, D, W), lambda i: (0, 0, 0)),
            pl.BlockSpec((L, D), lambda i: (0, 0)),
            pl.BlockSpec((2, D), lambda i: (0, 0)),
        ],
        out_specs=pl.BlockSpec((L, D, TB), lambda i: (0, 0, i)),
        out_shape=jax.ShapeDtypeStruct((L, D, B), jnp.float32),
        scratch_shapes=[
            pltpu.VMEM((L, D, TB), jnp.float32),
            pltpu.VMEM((D, TB), jnp.float32),
        ],
    )(colmod, syndrome_t, tiles, c0, val_emb)


def kernel(syndrome, stab_id, cycle_id, stab_emb, cycle_emb, val_emb):
    syndrome = syndrome.astype(jnp.int32)
    stab_id = stab_id.astype(jnp.int32)
    cycle_id = cycle_id.astype(jnp.int32)
    tiles, c0 = _sc_fetch(stab_id, cycle_id, stab_emb.T, cycle_emb, val_emb)
    colmod = stab_id % 128
    out_t = _tc_expand(syndrome.T, tiles, colmod, c0, val_emb)
    return jnp.transpose(out_t, (2, 0, 1))
